# Initial kernel scaffold; baseline (speedup 1.0000x reference)
import functools

import jax
import jax.numpy as jnp
from jax import lax
from jax.experimental import pallas as pl
from jax.experimental.pallas import tpu as pltpu
from jax.experimental.pallas import tpu_sc as plsc

N_NODES = 10000
D_SIG = 128
N_QUERY = 8192
NC, NS = 2, 16
NW = NC * NS
NPAD = 10016
ROWS_PER_TILE = NPAD // NS
CHUNK = 128
FCH = 64


def _mesh():
    return plsc.VectorSubcoreMesh(
        core_axis_name="c", subcore_axis_name="s", num_cores=NC, num_subcores=NS
    )


@functools.lru_cache(None)
def _scatter_kernel(e_pad, d):
    e_per_w = e_pad // NW
    nch = e_per_w // CHUNK

    def body(table, rows, cols, zrows, out, row_v, col_v, buf, acc, sem):
        cid = lax.axis_index("c")
        sid = lax.axis_index("s")
        wid = sid * NC + cid
        pltpu.sync_copy(zrows, acc.at[pl.ds(sid * ROWS_PER_TILE, ROWS_PER_TILE)])
        pltpu.sync_copy(rows.at[wid], row_v)
        pltpu.sync_copy(cols.at[wid], col_v)
        plsc.subcore_barrier()

        def step(j, carry):
            pltpu.async_copy(table.at[row_v.at[j]], buf, sem).wait()
            pltpu.sync_copy(buf, acc.at[col_v.at[j]], add=True)
            return carry

        lax.fori_loop(0, nch, step, 0)
        plsc.subcore_barrier()
        pltpu.sync_copy(
            acc.at[pl.ds(sid * ROWS_PER_TILE, ROWS_PER_TILE)],
            out.at[cid, pl.ds(sid * ROWS_PER_TILE, ROWS_PER_TILE)],
        )

    return pl.kernel(
        body,
        out_type=jax.ShapeDtypeStruct((NC, NPAD, d), jnp.float32),
        mesh=_mesh(),
        scratch_types=[
            pltpu.VMEM((nch, CHUNK), jnp.int32),
            pltpu.VMEM((nch, CHUNK), jnp.int32),
            pltpu.VMEM((CHUNK, d), jnp.float32),
            pltpu.VMEM_SHARED((NPAD, d), jnp.float32),
            pltpu.SemaphoreType.DMA,
        ],
    )


@functools.lru_cache(None)
def _final_kernel():
    e_per_w = N_QUERY // NW
    nch = e_per_w // FCH

    def body(a_t, t_t, x_t, g_t, e0s, e1s,
             o11, o12, o21, o22, os12, os21, od0, od1,
             e0_v, e1_v, a0b, a1b, t0b, t1b, x0b, x1b, g0b, g1b,
             v11b, v12b, v21b, v22b, vs12b, vs21b, vd0b, vd1b, sem):
        cid = lax.axis_index("c")
        sid = lax.axis_index("s")
        wid = sid * NC + cid
        pltpu.sync_copy(e0s.at[wid], e0_v)
        pltpu.sync_copy(e1s.at[wid], e1_v)

        def chunk_step(j, carry):
            pltpu.async_copy(a_t.at[e0_v.at[j]], a0b, sem).wait()
            pltpu.async_copy(a_t.at[e1_v.at[j]], a1b, sem).wait()
            pltpu.async_copy(t_t.at[e0_v.at[j]], t0b, sem).wait()
            pltpu.async_copy(t_t.at[e1_v.at[j]], t1b, sem).wait()
            pltpu.async_copy(x_t.at[e0_v.at[j]], x0b, sem).wait()
            pltpu.async_copy(x_t.at[e1_v.at[j]], x1b, sem).wait()
            pltpu.async_copy(g_t.at[e0_v.at[j]], g0b, sem).wait()
            pltpu.async_copy(g_t.at[e1_v.at[j]], g1b, sem).wait()

            def edge_step(i, c2):
                d0 = g0b[i, 0]
                d1 = g1b[i, 0]
                z = jnp.zeros((16,), jnp.float32)
                v11 = v12 = v21 = v22 = vs12 = vs21 = z
                for k in range(D_SIG // 16):
                    sl = pl.ds(k * 16, 16)
                    a0 = a0b[i, sl]
                    a1 = a1b[i, sl]
                    t0 = t0b[i, sl]
                    t1 = t1b[i, sl]
                    u0 = t0 - d0 * x0b[i, sl]
                    u1 = t1 - d1 * x1b[i, sl]
                    v11 = v11 + a0 * a1
                    v12 = v12 + a0 * t1
                    v21 = v21 + t0 * a1
                    v22 = v22 + u0 * u1
                    vs12 = vs12 + a0 * t0
                    vs21 = vs21 + a1 * t1
                v11b[i] = jnp.sum(v11)
                v12b[i] = jnp.sum(v12)
                v21b[i] = jnp.sum(v21)
                v22b[i] = jnp.sum(v22)
                vs12b[i] = jnp.sum(vs12)
                vs21b[i] = jnp.sum(vs21)
                vd0b[i] = d0
                vd1b[i] = d1
                return c2

            lax.fori_loop(0, FCH, edge_step, 0)
            base = wid * e_per_w + j * FCH
            pltpu.sync_copy(v11b, o11.at[pl.ds(base, FCH)])
            pltpu.sync_copy(v12b, o12.at[pl.ds(base, FCH)])
            pltpu.sync_copy(v21b, o21.at[pl.ds(base, FCH)])
            pltpu.sync_copy(v22b, o22.at[pl.ds(base, FCH)])
            pltpu.sync_copy(vs12b, os12.at[pl.ds(base, FCH)])
            pltpu.sync_copy(vs21b, os21.at[pl.ds(base, FCH)])
            pltpu.sync_copy(vd0b, od0.at[pl.ds(base, FCH)])
            pltpu.sync_copy(vd1b, od1.at[pl.ds(base, FCH)])
            return carry

        lax.fori_loop(0, nch, chunk_step, 0)

    q = jax.ShapeDtypeStruct((N_QUERY,), jnp.float32)
    return pl.kernel(
        body,
        out_type=(q,) * 8,
        mesh=_mesh(),
        scratch_types=[
            pltpu.VMEM((nch, FCH), jnp.int32),
            pltpu.VMEM((nch, FCH), jnp.int32),
        ]
        + [pltpu.VMEM((FCH, D_SIG), jnp.float32)] * 6
        + [pltpu.VMEM((FCH, 16), jnp.float32)] * 2
        + [pltpu.VMEM((FCH,), jnp.float32)] * 8
        + [pltpu.SemaphoreType.DMA],
    )


def _norm_body(x_ref, o_ref):
    x = x_ref[:, :]
    n = jnp.sqrt(jnp.sum(x * x, axis=1, keepdims=True))
    o_ref[:, :] = x / jnp.clip(n, 1e-12)


def _mask_body(p_ref, o_ref):
    o_ref[:, :] = ((p_ref[0] + p_ref[1]) > 0).astype(jnp.float32)


def _bfs_merge_body(p_ref, s_ref, cur_ref, snew_ref):
    r = ((p_ref[0] + p_ref[1]) > 0).astype(jnp.float32)
    cur_ref[:, :] = r
    snew_ref[:, :] = ((s_ref[:, :] + r) > 0).astype(jnp.float32)


def _yaug_body(s_ref, x_ref, o_ref):
    s = s_ref[:, 0:1]
    o_ref[:, :] = jnp.concatenate([x_ref[:, :] * s, s_ref[:, :]], axis=1)


def _merge_a_body(p_ref, s_ref, a_ref, m_ref, g_ref):
    tot = p_ref[0] + p_ref[1]
    a = tot[:, 0:D_SIG]
    a_ref[:, :] = a
    m_ref[:, :] = a * s_ref[:, 0:1]
    g_ref[:, :] = tot[:, D_SIG:D_SIG + 16]


def _merge_t_body(p_ref, t_ref):
    t_ref[:, :] = p_ref[0] + p_ref[1]


def _tc(body, out_shape, *args):
    return pl.pallas_call(body, out_shape=out_shape)(*args)


def _pad_edges(idx_g, idx_s, e_pad):
    pad = e_pad - idx_g.shape[0]
    if pad:
        fill = jnp.full((pad,), N_NODES, jnp.int32)
        idx_g = jnp.concatenate([idx_g, fill])
        idx_s = jnp.concatenate([idx_s, fill])
    nch = e_pad // NW // CHUNK
    return idx_g.reshape(NW, nch, CHUNK), idx_s.reshape(NW, nch, CHUNK)


def kernel(edges, edge_index, node_sig):
    col = edge_index[0]
    row = edge_index[1]
    e_pad = ((edge_index.shape[1] + NW * CHUNK - 1) // (NW * CHUNK)) * (NW * CHUNK)
    rows_g, cols_s = _pad_edges(row, col, e_pad)

    f32 = jnp.float32
    z16 = jnp.zeros((ROWS_PER_TILE, 16), f32)
    z128 = jnp.zeros((ROWS_PER_TILE, D_SIG), f32)
    z144 = jnp.zeros((ROWS_PER_TILE, D_SIG + 16), f32)

    x_pad = jnp.zeros((NPAD, D_SIG), f32).at[:N_NODES].set(node_sig)
    x_n = _tc(_norm_body, jax.ShapeDtypeStruct((NPAD, D_SIG), f32), x_pad)

    nodes = jnp.concatenate([edges[0], edges[1]]).astype(jnp.int32)
    onehot_t = jnp.zeros((8, 16), f32).at[0, 0].set(1.0)
    qg, qs = _pad_edges(jnp.zeros_like(nodes), nodes, nodes.shape[0])
    q_parts = _scatter_kernel(nodes.shape[0], 16)(onehot_t, qg, qs, z16)
    q_t = _tc(_mask_body, jax.ShapeDtypeStruct((NPAD, 16), f32), q_parts)

    r1_parts = _scatter_kernel(e_pad, 16)(q_t, rows_g, cols_s, z16)
    cur1_t, s1_t = _tc(
        _bfs_merge_body,
        (jax.ShapeDtypeStruct((NPAD, 16), f32),) * 2,
        r1_parts, q_t,
    )
    r2_parts = _scatter_kernel(e_pad, 16)(cur1_t, rows_g, cols_s, z16)
    _, s_t = _tc(
        _bfs_merge_body,
        (jax.ShapeDtypeStruct((NPAD, 16), f32),) * 2,
        r2_parts, s1_t,
    )

    y_aug = _tc(_yaug_body, jax.ShapeDtypeStruct((NPAD, D_SIG + 16), f32), s_t, x_n)
    a_parts = _scatter_kernel(e_pad, D_SIG + 16)(y_aug, rows_g, cols_s, z144)
    a_t, m_t, g_t = _tc(
        _merge_a_body,
        (
            jax.ShapeDtypeStruct((NPAD, D_SIG), f32),
            jax.ShapeDtypeStruct((NPAD, D_SIG), f32),
            jax.ShapeDtypeStruct((NPAD, 16), f32),
        ),
        a_parts, s_t,
    )

    t_parts = _scatter_kernel(e_pad, D_SIG)(m_t, rows_g, cols_s, z128)
    t_t = _tc(_merge_t_body, jax.ShapeDtypeStruct((NPAD, D_SIG), f32), t_parts)

    nq = edges.shape[1]
    nchq = nq // NW // FCH
    e0s = edges[0].astype(jnp.int32).reshape(NW, nchq, FCH)
    e1s = edges[1].astype(jnp.int32).reshape(NW, nchq, FCH)
    return _final_kernel()(a_t, t_t, x_n, g_t, e0s, e1s)


# speedup vs baseline: 16.8355x; 16.8355x over previous
import functools

import jax
import jax.numpy as jnp
from jax import lax
from jax.experimental import pallas as pl
from jax.experimental.pallas import tpu as pltpu
from jax.experimental.pallas import tpu_sc as plsc

N_NODES = 10000
D_SIG = 128
N_QUERY = 8192
NC, NS = 2, 16
NW = NC * NS
NPAD = 10112
ROWS_PER_TILE = NPAD // NS
NROW2 = NPAD // 128
CHUNK = 128
FCH = 64
JUNK = N_NODES


def _mesh():
    return plsc.VectorSubcoreMesh(
        core_axis_name="c", subcore_axis_name="s", num_cores=NC, num_subcores=NS
    )


_SC_PARAMS = pltpu.CompilerParams(needs_layout_passes=False)


def _wid():
    return lax.axis_index("s") * NC + lax.axis_index("c")


def _zero_2d(ref):
    def z(i, c):
        for k in range(8):
            ref[i, pl.ds(k * 16, 16)] = jnp.zeros((16,), jnp.float32)
        return c
    lax.fori_loop(0, NROW2, z, 0)


@functools.lru_cache(None)
def _vec_scatter_kernel(e_pad, ntab):
    e_per_w = e_pad // NW

    def body(*refs):
        tabs = refs[:ntab]
        rows, cols = refs[ntab], refs[ntab + 1]
        outs = refs[ntab + 2:ntab + 2 + ntab]
        scratch = refs[ntab + 2 + ntab:]
        tab_vs = scratch[:ntab]
        acc_vs = scratch[ntab:2 * ntab]
        row_v, col_v = scratch[2 * ntab], scratch[2 * ntab + 1]
        wid = _wid()
        for t in range(ntab):
            pltpu.sync_copy(tabs[t], tab_vs[t])
            _zero_2d(acc_vs[t])
        pltpu.sync_copy(rows.at[wid], row_v)
        pltpu.sync_copy(cols.at[wid], col_v)

        def step(i, c):
            r = row_v[pl.ds(i * 16, 16)]
            cc = col_v[pl.ds(i * 16, 16)]
            for t in range(ntab):
                v = plsc.load_gather(tab_vs[t], [r >> 7, r & 127])
                plsc.addupdate_scatter(acc_vs[t], [cc >> 7, cc & 127], v)
            return c

        lax.fori_loop(0, e_per_w // 16, step, 0)
        for t in range(ntab):
            pltpu.sync_copy(acc_vs[t], outs[t].at[wid])

    return pl.kernel(
        body,
        out_type=tuple(
            jax.ShapeDtypeStruct((NW, NROW2, 128), jnp.float32)
            for _ in range(ntab)
        ),
        mesh=_mesh(),
        compiler_params=_SC_PARAMS,
        scratch_types=[pltpu.VMEM((NROW2, 128), jnp.float32)] * (2 * ntab)
        + [
            pltpu.VMEM((e_per_w,), jnp.int32),
            pltpu.VMEM((e_per_w,), jnp.int32),
        ],
    )


@functools.lru_cache(None)
def _spmm_kernel(e_tot):
    e_per_w = e_tot // NW
    BLK = 2048
    nblk = e_per_w // BLK
    cpb = BLK // CHUNK
    nch = e_per_w // CHUNK

    def body(table, rows, cols, zrows, out, row_v, col_v, buf, acc, sem):
        cid = lax.axis_index("c")
        sid = lax.axis_index("s")
        wid = sid * NC + cid
        pltpu.sync_copy(zrows, acc.at[pl.ds(sid * ROWS_PER_TILE, ROWS_PER_TILE)])
        pltpu.sync_copy(cols.at[wid], col_v)
        plsc.subcore_barrier()

        def block_step(b, carry):
            pltpu.sync_copy(rows.at[wid, pl.ds(b * BLK, BLK)], row_v)

            def step(j, c):
                pltpu.async_copy(
                    table.at[row_v.at[pl.ds(j * CHUNK, CHUNK)]], buf, sem
                ).wait()
                pltpu.sync_copy(buf, acc.at[col_v.at[b * cpb + j]], add=True)
                return c

            lax.fori_loop(0, cpb, step, 0)
            return carry

        lax.fori_loop(0, nblk, block_step, 0)
        plsc.subcore_barrier()
        pltpu.sync_copy(
            acc.at[pl.ds(sid * ROWS_PER_TILE, ROWS_PER_TILE)],
            out.at[cid, pl.ds(sid * ROWS_PER_TILE, ROWS_PER_TILE)],
        )

    return pl.kernel(
        body,
        out_type=jax.ShapeDtypeStruct((NC, NPAD, D_SIG), jnp.float32),
        mesh=_mesh(),
        compiler_params=_SC_PARAMS,
        scratch_types=[
            pltpu.VMEM((BLK,), jnp.int32),
            pltpu.VMEM((nch, CHUNK), jnp.int32),
            pltpu.VMEM((CHUNK, D_SIG), jnp.float32),
            pltpu.VMEM_SHARED((NPAD, D_SIG), jnp.float32),
            pltpu.SemaphoreType.DMA,
        ],
    )


@functools.lru_cache(None)
def _deg_kernel(e_pad):
    e_per_w = e_pad // NW

    def body(s_t, rows, cols, out, rows_m, tab_v, acc_v, row_v, col_v):
        wid = _wid()
        pltpu.sync_copy(s_t, tab_v)
        _zero_2d(acc_v)
        pltpu.sync_copy(rows.at[wid], row_v)
        pltpu.sync_copy(cols.at[wid], col_v)

        def step(i, c):
            r = row_v[pl.ds(i * 16, 16)]
            cc = col_v[pl.ds(i * 16, 16)]
            v = plsc.load_gather(tab_v, [r >> 7, r & 127])
            plsc.addupdate_scatter(acc_v, [cc >> 7, cc & 127], v)
            row_v[pl.ds(i * 16, 16)] = jnp.where(v > 0.0, r, JUNK)
            return c

        lax.fori_loop(0, e_per_w // 16, step, 0)
        pltpu.sync_copy(acc_v, out.at[wid])
        pltpu.sync_copy(row_v, rows_m.at[wid])

    return pl.kernel(
        body,
        out_type=(
            jax.ShapeDtypeStruct((NW, NROW2, 128), jnp.float32),
            jax.ShapeDtypeStruct((NW, e_per_w), jnp.int32),
        ),
        mesh=_mesh(),
        compiler_params=_SC_PARAMS,
        scratch_types=[
            pltpu.VMEM((NROW2, 128), jnp.float32),
            pltpu.VMEM((NROW2, 128), jnp.float32),
            pltpu.VMEM((e_per_w,), jnp.int32),
            pltpu.VMEM((e_per_w,), jnp.int32),
        ],
    )


@functools.lru_cache(None)
def _final_kernel():
    e_per_w = N_QUERY // NW
    nch = e_per_w // FCH

    def body(a_t, t_t, x_t, deg, e0s, e1s,
             o11, o12, o21, o22, os12, os21, od0, od1,
             e0_v, e1_v, deg_v, a0b, a1b, t0b, t1b, x0b, x1b,
             v11b, v12b, v21b, v22b, vs12b, vs21b, vd0b, vd1b, sem):
        wid = _wid()
        pltpu.sync_copy(e0s.at[wid], e0_v)
        pltpu.sync_copy(e1s.at[wid], e1_v)
        pltpu.sync_copy(deg, deg_v)

        def chunk_step(j, carry):
            sj = pl.ds(j * FCH, FCH)
            pltpu.async_copy(a_t.at[e0_v.at[sj]], a0b, sem).wait()
            pltpu.async_copy(a_t.at[e1_v.at[sj]], a1b, sem).wait()
            pltpu.async_copy(t_t.at[e0_v.at[sj]], t0b, sem).wait()
            pltpu.async_copy(t_t.at[e1_v.at[sj]], t1b, sem).wait()
            pltpu.async_copy(x_t.at[e0_v.at[sj]], x0b, sem).wait()
            pltpu.async_copy(x_t.at[e1_v.at[sj]], x1b, sem).wait()

            def group_step(g, c2):
                i0 = jnp.arange(16, dtype=jnp.int32) + g * 16
                e0g = e0_v[pl.ds(j * FCH + g * 16, 16)]
                e1g = e1_v[pl.ds(j * FCH + g * 16, 16)]
                d0 = plsc.load_gather(deg_v, [e0g >> 7, e0g & 127])
                d1 = plsc.load_gather(deg_v, [e1g >> 7, e1g & 127])
                z = jnp.zeros((16,), jnp.float32)

                def feat_step(k, vs):
                    v11, v12, v21, v22, vs12, vs21 = vs
                    ck = jnp.zeros((16,), jnp.int32) + k
                    a0 = plsc.load_gather(a0b, [i0, ck])
                    a1 = plsc.load_gather(a1b, [i0, ck])
                    t0 = plsc.load_gather(t0b, [i0, ck])
                    t1 = plsc.load_gather(t1b, [i0, ck])
                    u0 = t0 - d0 * plsc.load_gather(x0b, [i0, ck])
                    u1 = t1 - d1 * plsc.load_gather(x1b, [i0, ck])
                    return (v11 + a0 * a1, v12 + a0 * t1, v21 + t0 * a1,
                            v22 + u0 * u1, vs12 + a0 * t0, vs21 + a1 * t1)

                v11, v12, v21, v22, vs12, vs21 = lax.fori_loop(
                    0, D_SIG, feat_step, (z, z, z, z, z, z))
                sl = pl.ds(g * 16, 16)
                v11b[sl] = v11
                v12b[sl] = v12
                v21b[sl] = v21
                v22b[sl] = v22
                vs12b[sl] = vs12
                vs21b[sl] = vs21
                vd0b[sl] = d0
                vd1b[sl] = d1
                return c2

            lax.fori_loop(0, FCH // 16, group_step, 0)
            base = wid * e_per_w + j * FCH
            pltpu.sync_copy(v11b, o11.at[pl.ds(base, FCH)])
            pltpu.sync_copy(v12b, o12.at[pl.ds(base, FCH)])
            pltpu.sync_copy(v21b, o21.at[pl.ds(base, FCH)])
            pltpu.sync_copy(v22b, o22.at[pl.ds(base, FCH)])
            pltpu.sync_copy(vs12b, os12.at[pl.ds(base, FCH)])
            pltpu.sync_copy(vs21b, os21.at[pl.ds(base, FCH)])
            pltpu.sync_copy(vd0b, od0.at[pl.ds(base, FCH)])
            pltpu.sync_copy(vd1b, od1.at[pl.ds(base, FCH)])
            return carry

        lax.fori_loop(0, nch, chunk_step, 0)

    q = jax.ShapeDtypeStruct((N_QUERY,), jnp.float32)
    return pl.kernel(
        body,
        out_type=(q,) * 8,
        mesh=_mesh(),
        compiler_params=_SC_PARAMS,
        scratch_types=[
            pltpu.VMEM((e_per_w,), jnp.int32),
            pltpu.VMEM((e_per_w,), jnp.int32),
            pltpu.VMEM((NROW2, 128), jnp.float32),
        ]
        + [pltpu.VMEM((FCH, D_SIG), jnp.float32)] * 6
        + [pltpu.VMEM((FCH,), jnp.float32)] * 8
        + [pltpu.SemaphoreType.DMA],
    )


def _norm_body(x_ref, o_ref):
    x = x_ref[:, :]
    n = jnp.sqrt(jnp.sum(x * x, axis=1, keepdims=True))
    o_ref[:, :] = x / jnp.clip(n, 1e-12)


def _mask_body(p_ref, o_ref):
    o_ref[...] = (jnp.sum(p_ref[...], axis=0) > 0).astype(jnp.float32)


def _bfs_merge_body(p_ref, s_ref, cur_ref, snew_ref):
    r = (jnp.sum(p_ref[...], axis=0) > 0).astype(jnp.float32)
    cur_ref[...] = r
    snew_ref[...] = ((s_ref[...] + r) > 0).astype(jnp.float32)


def _final_merge_body(p_ref, pd_ref, s_ref, s2_ref, deg_ref):
    r = (jnp.sum(p_ref[...], axis=0) > 0).astype(jnp.float32)
    s2_ref[...] = ((s_ref[...] + r) > 0).astype(jnp.float32)
    deg_ref[...] = jnp.sum(pd_ref[...], axis=0)


def _merge2_body(p_ref, o_ref):
    o_ref[:, :] = p_ref[0] + p_ref[1]


def _tc(body, out_shape, *args):
    return pl.pallas_call(body, out_shape=out_shape)(*args)


def _pad_pair(idx_g, idx_s, e_pad):
    pad = e_pad - idx_g.shape[0]
    if pad:
        fill = jnp.full((pad,), JUNK, jnp.int32)
        idx_g = jnp.concatenate([idx_g, fill])
        idx_s = jnp.concatenate([idx_s, fill])
    return idx_g.reshape(NW, -1), idx_s.reshape(NW, -1)


def kernel(edges, edge_index, node_sig):
    col = edge_index[0]
    row = edge_index[1]
    e_pad = ((edge_index.shape[1] + NW * CHUNK - 1) // (NW * CHUNK)) * (NW * CHUNK)
    rows_g, cols_s = _pad_pair(row, col, e_pad)

    f32 = jnp.float32
    zrows = jnp.zeros((ROWS_PER_TILE, D_SIG), f32)
    npad_shape = jax.ShapeDtypeStruct((NROW2, 128), f32)

    x_pad = jnp.zeros((NPAD, D_SIG), f32).at[:N_NODES].set(node_sig)
    x_n = _tc(_norm_body, jax.ShapeDtypeStruct((NPAD, D_SIG), f32), x_pad)

    nodes = jnp.concatenate([edges[0], edges[1]]).astype(jnp.int32)
    ones_t = jnp.ones((NROW2, 128), f32)
    qg, qs = _pad_pair(jnp.zeros_like(nodes), nodes, nodes.shape[0])
    (q_parts,) = _vec_scatter_kernel(nodes.shape[0], 1)(ones_t, qg, qs)
    q_t = _tc(_mask_body, npad_shape, q_parts)

    (r1_parts,) = _vec_scatter_kernel(e_pad, 1)(q_t, rows_g, cols_s)
    cur1_t, s1_t = _tc(_bfs_merge_body, (npad_shape,) * 2, r1_parts, q_t)
    (r2_parts,) = _vec_scatter_kernel(e_pad, 1)(cur1_t, rows_g, cols_s)
    _, s_t = _tc(_bfs_merge_body, (npad_shape,) * 2, r2_parts, s1_t)
    deg_parts, rows_m = _deg_kernel(e_pad)(s_t, rows_g, cols_s)
    deg_t = _tc(_mask_sum_body, npad_shape, deg_parts)

    BLK = 2048
    junk = jnp.full((NW, BLK), JUNK, jnp.int32)
    rows_mt = jnp.concatenate([rows_m, junk], axis=1)
    cols_t = jnp.concatenate([cols_s, junk], axis=1)
    e_tot = e_pad + NW * BLK
    nch_t = e_tot // NW // CHUNK
    cols_t3 = cols_t.reshape(NW, nch_t, CHUNK)

    a_parts = _spmm_kernel(e_tot)(x_n, rows_mt, cols_t3, zrows)
    a_t = _tc(_merge2_body, jax.ShapeDtypeStruct((NPAD, D_SIG), f32), a_parts)

    t_parts = _spmm_kernel(e_tot)(a_t, rows_mt, cols_t3, zrows)
    t_t = _tc(_merge2_body, jax.ShapeDtypeStruct((NPAD, D_SIG), f32), t_parts)

    nq = edges.shape[1]
    e0s = edges[0].astype(jnp.int32).reshape(NW, nq // NW)
    e1s = edges[1].astype(jnp.int32).reshape(NW, nq // NW)
    return _final_kernel()(a_t, t_t, x_n, deg_t, e0s, e1s)


def _mask_sum_body(p_ref, o_ref):
    o_ref[...] = jnp.sum(p_ref[...], axis=0)
